# baseline (speedup 1.0000x reference)
import jax
import jax.numpy as jnp
from jax.experimental import pallas as pl

_B = 4
_P = 576
_S = _P + 1
_D = 768
_H = 12
_HD = _D // _H
_K = 32
_TEMP = 0.1
_SCALE = _HD ** (-0.5)
_MLP = 4 * _D

_QB = 128
_SP = 640

_F32 = jnp.float32
_BF16 = jnp.bfloat16


def _ln(x, g, b):
    mu = jnp.mean(x, axis=1, keepdims=True)
    var = jnp.mean((x - mu) ** 2, axis=1, keepdims=True)
    return (x - mu) / jnp.sqrt(var + 1e-5) * g + b


def _dot_t(a, b):
    return jax.lax.dot_general(a, b, (((1,), (1,)), ((), ())),
                               preferred_element_type=_F32)


def _stage_a(x_ref, w_ref, bb_ref, g_ref, b_ref,
             qn_ref, kn_ref, q_ref, k_ref, v_ref):
    i = pl.program_id(1)
    rows = i * _QB + jax.lax.broadcasted_iota(jnp.int32, (_QB, 1), 0)
    xn = _ln(x_ref[...], g_ref[...], b_ref[...])
    xn = jnp.where(rows < _S, xn, 0.0)
    proj = jnp.dot(xn.astype(_BF16), w_ref[...],
                   preferred_element_type=_F32) + bb_ref[...]
    q = proj[:, :_D]
    qn_ref[...] = (q / jnp.maximum(
        jnp.sqrt(jnp.sum(q * q, axis=1, keepdims=True)), 1e-12)).astype(_BF16)
    k = proj[:, _D:2 * _D]
    kn_ref[...] = (k / jnp.maximum(
        jnp.sqrt(jnp.sum(k * k, axis=1, keepdims=True)), 1e-12)).astype(_BF16)
    q_ref[...] = proj[:, 2 * _D:3 * _D].astype(_BF16)
    k_ref[...] = proj[:, 3 * _D:4 * _D].astype(_BF16)
    v_ref[...] = proj[:, 4 * _D:].astype(_BF16)


def _stage_bc(qn_ref, kn_ref, pos_ref, q_ref, k_ref, v_ref, out_ref):
    i = pl.program_id(1)
    rows = i * _QB + jax.lax.broadcasted_iota(jnp.int32, (_QB, _SP), 0)
    cols = jax.lax.broadcasted_iota(jnp.int32, (_QB, _SP), 1)
    rs = _dot_t(qn_ref[...], kn_ref[...]) + pos_ref[...]
    valid = (cols >= 1) & (cols < _S) & (cols != rows)
    st = jnp.where(valid, rs * (1.0 / _TEMP), -1e30)
    m0 = jnp.max(st, axis=1, keepdims=True)
    lo = m0 - 23.0
    hi = m0
    for _ in range(16):
        t = 0.5 * (lo + hi)
        cnt = jnp.sum(jnp.where(st >= t, 1.0, 0.0), axis=1, keepdims=True)
        pred = cnt >= float(_K)
        lo = jnp.where(pred, t, lo)
        hi = jnp.where(pred, hi, t)
    sel = st >= lo
    e = jnp.where(sel, jnp.exp(st - m0), 0.0)
    z = jnp.sum(e, axis=1, keepdims=True)
    bias = jnp.where(sel, jnp.maximum(st - m0 - jnp.log(z), -10.0), -1e9)
    bias = jnp.where(rows == 0, jnp.where(cols < _S, 0.0, -1e9), bias)
    bias = jnp.where(rows >= _S, 0.0, bias)

    q = q_ref[...]
    for h in range(_H):
        sl = slice(h * _HD, (h + 1) * _HD)
        s = _dot_t(q[:, sl], k_ref[:, sl]) * _SCALE + bias
        m = jnp.max(s, axis=1, keepdims=True)
        p = jnp.exp(s - m)
        zz = jnp.sum(p, axis=1, keepdims=True)
        out_ref[:, sl] = (jnp.dot(p.astype(_BF16), v_ref[:, sl],
                                  preferred_element_type=_F32) / zz
                          ).astype(_BF16)


def _stage_d(ao_ref, x_ref, wp_ref, bp_ref, g2_ref, b2_ref,
             w1_ref, b1_ref, w2_ref, bb2_ref, out_ref):
    h = jnp.dot(ao_ref[...], wp_ref[...],
                preferred_element_type=_F32) + bp_ref[...] + x_ref[...]
    hn = _ln(h, g2_ref[...], b2_ref[...])
    u = jnp.dot(hn.astype(_BF16), w1_ref[...],
                preferred_element_type=_F32) + b1_ref[...]
    gelu = 0.5 * u * (1.0 + jax.lax.erf(u * (2.0 ** -0.5)))
    out_ref[...] = h + jnp.dot(gelu.astype(_BF16), w2_ref[...],
                               preferred_element_type=_F32) + bb2_ref[...]


def kernel(x, Wq, bq, Wk, bk, pos_bias, Wqkv, bqkv, Wproj, bproj,
           ln1_g, ln1_b, ln2_g, ln2_b, W1, b1, W2, b2):
    row2 = lambda a: a.reshape(1, -1)
    full = lambda shape: pl.BlockSpec(shape, lambda *_: (0,) * len(shape))
    rowblk = pl.BlockSpec((None, _QB, _D), lambda b, i: (b, i, 0))
    seqblk = pl.BlockSpec((None, _SP, _D), lambda b, i: (b, 0, 0))
    bf3 = jax.ShapeDtypeStruct((_B, _SP, _D), _BF16)
    grid = (_B, _SP // _QB)

    Wall = jnp.concatenate([Wq, Wk, Wqkv], axis=1).astype(_BF16)
    ball = jnp.concatenate([bq, bk, bqkv]).reshape(1, -1)
    pos_pad = jnp.pad(pos_bias, ((1, _SP - _S), (1, _SP - _S)))

    qn, kn, Q, K, V = pl.pallas_call(
        _stage_a,
        grid=grid,
        in_specs=[
            rowblk,
            full((_D, 5 * _D)), full((1, 5 * _D)),
            full((1, _D)), full((1, _D)),
        ],
        out_specs=[rowblk] * 5,
        out_shape=[bf3] * 5,
    )(x, Wall, ball, row2(ln1_g), row2(ln1_b))

    att = pl.pallas_call(
        _stage_bc,
        grid=grid,
        in_specs=[
            rowblk,
            seqblk,
            pl.BlockSpec((_QB, _SP), lambda b, i: (i, 0)),
            rowblk,
            seqblk,
            seqblk,
        ],
        out_specs=rowblk,
        out_shape=jax.ShapeDtypeStruct((_B, _SP, _D), _BF16),
    )(qn, kn, pos_pad, Q, K, V)

    out = pl.pallas_call(
        _stage_d,
        grid=grid,
        in_specs=[
            rowblk,
            rowblk,
            full((_D, _D)), full((1, _D)),
            full((1, _D)), full((1, _D)),
            full((_D, _MLP)), full((1, _MLP)),
            full((_MLP, _D)), full((1, _D)),
        ],
        out_specs=rowblk,
        out_shape=jax.ShapeDtypeStruct((_B, _S, _D), _F32),
    )(att, x, Wproj.astype(_BF16), row2(bproj), row2(ln2_g), row2(ln2_b),
      W1.astype(_BF16), row2(b1), W2.astype(_BF16), row2(b2))

    return out


# speedup vs baseline: 1.1175x; 1.1175x over previous
import jax
import jax.numpy as jnp
from jax.experimental import pallas as pl

_B = 4
_P = 576
_S = _P + 1
_D = 768
_H = 12
_HD = _D // _H
_K = 32
_TEMP = 0.1
_SCALE = _HD ** (-0.5)
_MLP = 4 * _D

_QB = 160
_SP = 640

_F32 = jnp.float32
_BF16 = jnp.bfloat16


def _ln(x, g, b):
    mu = jnp.mean(x, axis=1, keepdims=True)
    var = jnp.mean((x - mu) ** 2, axis=1, keepdims=True)
    return (x - mu) / jnp.sqrt(var + 1e-5) * g + b


def _dot_t(a, b):
    return jax.lax.dot_general(a, b, (((1,), (1,)), ((), ())),
                               preferred_element_type=_F32)


def _stage_a(x_ref, w_ref, bb_ref, g_ref, b_ref,
             qn_ref, kn_ref, q_ref, k_ref, v_ref):
    i = pl.program_id(1)
    rows = i * _QB + jax.lax.broadcasted_iota(jnp.int32, (_QB, 1), 0)
    xn = _ln(x_ref[...], g_ref[...], b_ref[...])
    xn = jnp.where(rows < _S, xn, 0.0)
    proj = jnp.dot(xn.astype(_BF16), w_ref[...],
                   preferred_element_type=_F32) + bb_ref[...]
    q = proj[:, :_D]
    qn_ref[...] = (q / jnp.maximum(
        jnp.sqrt(jnp.sum(q * q, axis=1, keepdims=True)), 1e-12)).astype(_BF16)
    k = proj[:, _D:2 * _D]
    kn_ref[...] = (k / jnp.maximum(
        jnp.sqrt(jnp.sum(k * k, axis=1, keepdims=True)), 1e-12)).astype(_BF16)
    q_ref[...] = proj[:, 2 * _D:3 * _D].astype(_BF16)
    k_ref[...] = proj[:, 3 * _D:4 * _D].astype(_BF16)
    v_ref[...] = proj[:, 4 * _D:].astype(_BF16)


def _stage_bc(qn_ref, kn_ref, pos_ref, q_ref, k_ref, v_ref, out_ref):
    i = pl.program_id(1)
    rows = i * _QB + jax.lax.broadcasted_iota(jnp.int32, (_QB, _SP), 0)
    cols = jax.lax.broadcasted_iota(jnp.int32, (_QB, _SP), 1)
    rs = _dot_t(qn_ref[...], kn_ref[...]) + pos_ref[...]
    valid = (cols >= 1) & (cols < _S) & (cols != rows)
    st = jnp.where(valid, rs * (1.0 / _TEMP), -1e30)
    m0 = jnp.max(st, axis=1, keepdims=True)
    lo = m0 - 23.0
    hi = m0
    for _ in range(16):
        t = 0.5 * (lo + hi)
        cnt = jnp.sum(jnp.where(st >= t, 1.0, 0.0), axis=1, keepdims=True)
        pred = cnt >= float(_K)
        lo = jnp.where(pred, t, lo)
        hi = jnp.where(pred, hi, t)
    sel = st >= lo
    e = jnp.where(sel, jnp.exp(st - m0), 0.0)
    z = jnp.sum(e, axis=1, keepdims=True)
    bias = jnp.where(sel, jnp.maximum(st - m0 - jnp.log(z), -10.0), -1e9)
    bias = jnp.where(rows == 0, jnp.where(cols < _S, 0.0, -1e9), bias)
    bias = jnp.where(rows >= _S, 0.0, bias)

    q = q_ref[...]
    for h in range(_H):
        sl = slice(h * _HD, (h + 1) * _HD)
        s = _dot_t(q[:, sl], k_ref[:, sl]) * _SCALE + bias
        m = jnp.max(s, axis=1, keepdims=True)
        p = jnp.exp(s - m)
        zz = jnp.sum(p, axis=1, keepdims=True)
        out_ref[:, sl] = (jnp.dot(p.astype(_BF16), v_ref[:, sl],
                                  preferred_element_type=_F32) / zz
                          ).astype(_BF16)


def _stage_d(ao_ref, x_ref, wp_ref, bp_ref, g2_ref, b2_ref,
             w1_ref, b1_ref, w2_ref, bb2_ref, out_ref):
    h = jnp.dot(ao_ref[...], wp_ref[...],
                preferred_element_type=_F32) + bp_ref[...] + x_ref[...]
    hn = _ln(h, g2_ref[...], b2_ref[...])
    u = jnp.dot(hn.astype(_BF16), w1_ref[...],
                preferred_element_type=_F32) + b1_ref[...]
    gelu = 0.5 * u * (1.0 + jax.lax.erf(u * (2.0 ** -0.5)))
    out_ref[...] = h + jnp.dot(gelu.astype(_BF16), w2_ref[...],
                               preferred_element_type=_F32) + bb2_ref[...]


def kernel(x, Wq, bq, Wk, bk, pos_bias, Wqkv, bqkv, Wproj, bproj,
           ln1_g, ln1_b, ln2_g, ln2_b, W1, b1, W2, b2):
    row2 = lambda a: a.reshape(1, -1)
    full = lambda shape: pl.BlockSpec(shape, lambda *_: (0,) * len(shape))
    rowblk = pl.BlockSpec((None, _QB, _D), lambda b, i: (b, i, 0))
    seqblk = pl.BlockSpec((None, _SP, _D), lambda b, i: (b, 0, 0))
    bf3 = jax.ShapeDtypeStruct((_B, _SP, _D), _BF16)
    grid = (_B, _SP // _QB)

    Wall = jnp.concatenate([Wq, Wk, Wqkv], axis=1).astype(_BF16)
    ball = jnp.concatenate([bq, bk, bqkv]).reshape(1, -1)
    pos_pad = jnp.pad(pos_bias, ((1, _SP - _S), (1, _SP - _S)))

    qn, kn, Q, K, V = pl.pallas_call(
        _stage_a,
        grid=grid,
        in_specs=[
            rowblk,
            full((_D, 5 * _D)), full((1, 5 * _D)),
            full((1, _D)), full((1, _D)),
        ],
        out_specs=[rowblk] * 5,
        out_shape=[bf3] * 5,
    )(x, Wall, ball, row2(ln1_g), row2(ln1_b))

    att = pl.pallas_call(
        _stage_bc,
        grid=grid,
        in_specs=[
            rowblk,
            seqblk,
            pl.BlockSpec((_QB, _SP), lambda b, i: (i, 0)),
            rowblk,
            seqblk,
            seqblk,
        ],
        out_specs=rowblk,
        out_shape=jax.ShapeDtypeStruct((_B, _SP, _D), _BF16),
    )(qn, kn, pos_pad, Q, K, V)

    out = pl.pallas_call(
        _stage_d,
        grid=grid,
        in_specs=[
            rowblk,
            rowblk,
            full((_D, _D)), full((1, _D)),
            full((1, _D)), full((1, _D)),
            full((_D, _MLP)), full((1, _MLP)),
            full((_MLP, _D)), full((1, _D)),
        ],
        out_specs=rowblk,
        out_shape=jax.ShapeDtypeStruct((_B, _S, _D), _F32),
    )(att, x, Wproj.astype(_BF16), row2(bproj), row2(ln2_g), row2(ln2_b),
      W1.astype(_BF16), row2(b1), W2.astype(_BF16), row2(b2))

    return out


# speedup vs baseline: 1.4097x; 1.2615x over previous
import jax
import jax.numpy as jnp
from jax.experimental import pallas as pl

_B = 4
_P = 576
_S = _P + 1
_D = 768
_H = 12
_HD = _D // _H
_K = 32
_TEMP = 0.1
_SCALE = _HD ** (-0.5)
_MLP = 4 * _D

_QB = 320
_SP = 640

_F32 = jnp.float32
_BF16 = jnp.bfloat16


def _ln(x, g, b):
    mu = jnp.mean(x, axis=1, keepdims=True)
    var = jnp.mean((x - mu) ** 2, axis=1, keepdims=True)
    return (x - mu) / jnp.sqrt(var + 1e-5) * g + b


def _dot_t(a, b):
    return jax.lax.dot_general(a, b, (((1,), (1,)), ((), ())),
                               preferred_element_type=_F32)


def _stage_a(x_ref, w_ref, bb_ref, g_ref, b_ref,
             qn_ref, kn_ref, q_ref, k_ref, v_ref):
    i = pl.program_id(1)
    rows = i * _QB + jax.lax.broadcasted_iota(jnp.int32, (_QB, 1), 0)
    xn = _ln(x_ref[...], g_ref[...], b_ref[...])
    xn = jnp.where(rows < _S, xn, 0.0)
    proj = jnp.dot(xn.astype(_BF16), w_ref[...],
                   preferred_element_type=_F32) + bb_ref[...]
    q = proj[:, :_D]
    qn_ref[...] = (q / jnp.maximum(
        jnp.sqrt(jnp.sum(q * q, axis=1, keepdims=True)), 1e-12)).astype(_BF16)
    k = proj[:, _D:2 * _D]
    kn_ref[...] = (k / jnp.maximum(
        jnp.sqrt(jnp.sum(k * k, axis=1, keepdims=True)), 1e-12)).astype(_BF16)
    q_ref[...] = proj[:, 2 * _D:3 * _D].astype(_BF16)
    k_ref[...] = proj[:, 3 * _D:4 * _D].astype(_BF16)
    v_ref[...] = proj[:, 4 * _D:].astype(_BF16)


def _stage_bc(qn_ref, kn_ref, pos_ref, q_ref, k_ref, v_ref, out_ref):
    i = pl.program_id(1)
    rows = i * _QB + jax.lax.broadcasted_iota(jnp.int32, (_QB, _SP), 0)
    cols = jax.lax.broadcasted_iota(jnp.int32, (_QB, _SP), 1)
    rs = _dot_t(qn_ref[...], kn_ref[...]) + pos_ref[...]
    valid = (cols >= 1) & (cols < _S) & (cols != rows)
    st = jnp.where(valid, rs * (1.0 / _TEMP), -1e30)
    m0 = jnp.max(st, axis=1, keepdims=True)
    lo = m0 - 23.0
    hi = m0
    for _ in range(16):
        t = 0.5 * (lo + hi)
        cnt = jnp.sum(jnp.where(st >= t, 1.0, 0.0), axis=1, keepdims=True)
        pred = cnt >= float(_K)
        lo = jnp.where(pred, t, lo)
        hi = jnp.where(pred, hi, t)
    sel = st >= lo
    e = jnp.where(sel, jnp.exp(st - m0), 0.0)
    z = jnp.sum(e, axis=1, keepdims=True)
    bias = jnp.where(sel, jnp.maximum(st - m0 - jnp.log(z), -10.0), -1e9)
    bias = jnp.where(rows == 0, jnp.where(cols < _S, 0.0, -1e9), bias)
    bias = jnp.where(rows >= _S, 0.0, bias)

    q = q_ref[...]
    for h in range(_H):
        sl = slice(h * _HD, (h + 1) * _HD)
        s = _dot_t(q[:, sl], k_ref[:, sl]) * _SCALE + bias
        m = jnp.max(s, axis=1, keepdims=True)
        p = jnp.exp(s - m)
        zz = jnp.sum(p, axis=1, keepdims=True)
        out_ref[:, sl] = (jnp.dot(p.astype(_BF16), v_ref[:, sl],
                                  preferred_element_type=_F32) / zz
                          ).astype(_BF16)


def _stage_d(ao_ref, x_ref, wp_ref, bp_ref, g2_ref, b2_ref,
             w1_ref, b1_ref, w2_ref, bb2_ref, out_ref):
    h = jnp.dot(ao_ref[...], wp_ref[...],
                preferred_element_type=_F32) + bp_ref[...] + x_ref[...]
    hn = _ln(h, g2_ref[...], b2_ref[...])
    u = jnp.dot(hn.astype(_BF16), w1_ref[...],
                preferred_element_type=_F32) + b1_ref[...]
    gelu = 0.5 * u * (1.0 + jax.lax.erf(u * (2.0 ** -0.5)))
    out_ref[...] = h + jnp.dot(gelu.astype(_BF16), w2_ref[...],
                               preferred_element_type=_F32) + bb2_ref[...]


def kernel(x, Wq, bq, Wk, bk, pos_bias, Wqkv, bqkv, Wproj, bproj,
           ln1_g, ln1_b, ln2_g, ln2_b, W1, b1, W2, b2):
    row2 = lambda a: a.reshape(1, -1)
    full = lambda shape: pl.BlockSpec(shape, lambda *_: (0,) * len(shape))
    rowblk = pl.BlockSpec((None, _QB, _D), lambda b, i: (b, i, 0))
    seqblk = pl.BlockSpec((None, _SP, _D), lambda b, i: (b, 0, 0))
    bf3 = jax.ShapeDtypeStruct((_B, _SP, _D), _BF16)
    grid = (_B, _SP // _QB)

    Wall = jnp.concatenate([Wq, Wk, Wqkv], axis=1).astype(_BF16)
    ball = jnp.concatenate([bq, bk, bqkv]).reshape(1, -1)
    pos_pad = jnp.pad(pos_bias, ((1, _SP - _S), (1, _SP - _S)))

    qn, kn, Q, K, V = pl.pallas_call(
        _stage_a,
        grid=grid,
        in_specs=[
            rowblk,
            full((_D, 5 * _D)), full((1, 5 * _D)),
            full((1, _D)), full((1, _D)),
        ],
        out_specs=[rowblk] * 5,
        out_shape=[bf3] * 5,
    )(x, Wall, ball, row2(ln1_g), row2(ln1_b))

    att = pl.pallas_call(
        _stage_bc,
        grid=grid,
        in_specs=[
            rowblk,
            seqblk,
            pl.BlockSpec((_QB, _SP), lambda b, i: (i, 0)),
            rowblk,
            seqblk,
            seqblk,
        ],
        out_specs=rowblk,
        out_shape=jax.ShapeDtypeStruct((_B, _SP, _D), _BF16),
    )(qn, kn, pos_pad, Q, K, V)

    out = pl.pallas_call(
        _stage_d,
        grid=grid,
        in_specs=[
            rowblk,
            rowblk,
            full((_D, _D)), full((1, _D)),
            full((1, _D)), full((1, _D)),
            full((_D, _MLP)), full((1, _MLP)),
            full((_MLP, _D)), full((1, _D)),
        ],
        out_specs=rowblk,
        out_shape=jax.ShapeDtypeStruct((_B, _S, _D), _F32),
    )(att, x, Wproj.astype(_BF16), row2(bproj), row2(ln2_g), row2(ln2_b),
      W1.astype(_BF16), row2(b1), W2.astype(_BF16), row2(b2))

    return out


# speedup vs baseline: 1.5333x; 1.0877x over previous
import jax
import jax.numpy as jnp
from jax.experimental import pallas as pl

_B = 4
_P = 576
_S = _P + 1
_D = 768
_H = 12
_HD = _D // _H
_K = 32
_TEMP = 0.1
_SCALE = _HD ** (-0.5)
_MLP = 4 * _D

_QB = 640
_SP = 640

_F32 = jnp.float32
_BF16 = jnp.bfloat16


def _ln(x, g, b):
    mu = jnp.mean(x, axis=1, keepdims=True)
    var = jnp.mean((x - mu) ** 2, axis=1, keepdims=True)
    return (x - mu) / jnp.sqrt(var + 1e-5) * g + b


def _dot_t(a, b):
    return jax.lax.dot_general(a, b, (((1,), (1,)), ((), ())),
                               preferred_element_type=_F32)


def _stage_a(x_ref, w_ref, bb_ref, g_ref, b_ref,
             qn_ref, kn_ref, q_ref, k_ref, v_ref):
    i = pl.program_id(1)
    rows = i * _QB + jax.lax.broadcasted_iota(jnp.int32, (_QB, 1), 0)
    xn = _ln(x_ref[...], g_ref[...], b_ref[...])
    xn = jnp.where(rows < _S, xn, 0.0)
    proj = jnp.dot(xn.astype(_BF16), w_ref[...],
                   preferred_element_type=_F32) + bb_ref[...]
    q = proj[:, :_D]
    qn_ref[...] = (q / jnp.maximum(
        jnp.sqrt(jnp.sum(q * q, axis=1, keepdims=True)), 1e-12)).astype(_BF16)
    k = proj[:, _D:2 * _D]
    kn_ref[...] = (k / jnp.maximum(
        jnp.sqrt(jnp.sum(k * k, axis=1, keepdims=True)), 1e-12)).astype(_BF16)
    q_ref[...] = proj[:, 2 * _D:3 * _D].astype(_BF16)
    k_ref[...] = proj[:, 3 * _D:4 * _D].astype(_BF16)
    v_ref[...] = proj[:, 4 * _D:].astype(_BF16)


def _stage_bc(qn_ref, kn_ref, pos_ref, q_ref, k_ref, v_ref, out_ref):
    i = pl.program_id(1)
    rows = i * _QB + jax.lax.broadcasted_iota(jnp.int32, (_QB, _SP), 0)
    cols = jax.lax.broadcasted_iota(jnp.int32, (_QB, _SP), 1)
    rs = _dot_t(qn_ref[...], kn_ref[...]) + pos_ref[...]
    valid = (cols >= 1) & (cols < _S) & (cols != rows)
    st = jnp.where(valid, rs * (1.0 / _TEMP), -1e30)
    m0 = jnp.max(st, axis=1, keepdims=True)
    lo = m0 - 23.0
    hi = m0
    for _ in range(16):
        t = 0.5 * (lo + hi)
        cnt = jnp.sum(jnp.where(st >= t, 1.0, 0.0), axis=1, keepdims=True)
        pred = cnt >= float(_K)
        lo = jnp.where(pred, t, lo)
        hi = jnp.where(pred, hi, t)
    sel = st >= lo
    e = jnp.where(sel, jnp.exp(st - m0), 0.0)
    z = jnp.sum(e, axis=1, keepdims=True)
    bias = jnp.where(sel, jnp.maximum(st - m0 - jnp.log(z), -10.0), -1e9)
    bias = jnp.where(rows == 0, jnp.where(cols < _S, 0.0, -1e9), bias)
    bias = jnp.where(rows >= _S, 0.0, bias)

    q = q_ref[...]
    for h in range(_H):
        sl = slice(h * _HD, (h + 1) * _HD)
        s = _dot_t(q[:, sl], k_ref[:, sl]) * _SCALE + bias
        m = jnp.max(s, axis=1, keepdims=True)
        p = jnp.exp(s - m)
        zz = jnp.sum(p, axis=1, keepdims=True)
        out_ref[:, sl] = (jnp.dot(p.astype(_BF16), v_ref[:, sl],
                                  preferred_element_type=_F32) / zz
                          ).astype(_BF16)


def _stage_d(ao_ref, x_ref, wp_ref, bp_ref, g2_ref, b2_ref,
             w1_ref, b1_ref, w2_ref, bb2_ref, out_ref):
    h = jnp.dot(ao_ref[...], wp_ref[...],
                preferred_element_type=_F32) + bp_ref[...] + x_ref[...]
    hn = _ln(h, g2_ref[...], b2_ref[...])
    u = jnp.dot(hn.astype(_BF16), w1_ref[...],
                preferred_element_type=_F32) + b1_ref[...]
    gelu = 0.5 * u * (1.0 + jax.lax.erf(u * (2.0 ** -0.5)))
    out_ref[...] = h + jnp.dot(gelu.astype(_BF16), w2_ref[...],
                               preferred_element_type=_F32) + bb2_ref[...]


def kernel(x, Wq, bq, Wk, bk, pos_bias, Wqkv, bqkv, Wproj, bproj,
           ln1_g, ln1_b, ln2_g, ln2_b, W1, b1, W2, b2):
    row2 = lambda a: a.reshape(1, -1)
    full = lambda shape: pl.BlockSpec(shape, lambda *_: (0,) * len(shape))
    rowblk = pl.BlockSpec((None, _QB, _D), lambda b, i: (b, i, 0))
    seqblk = pl.BlockSpec((None, _SP, _D), lambda b, i: (b, 0, 0))
    bf3 = jax.ShapeDtypeStruct((_B, _SP, _D), _BF16)
    grid = (_B, _SP // _QB)

    Wall = jnp.concatenate([Wq, Wk, Wqkv], axis=1).astype(_BF16)
    ball = jnp.concatenate([bq, bk, bqkv]).reshape(1, -1)
    pos_pad = jnp.pad(pos_bias, ((1, _SP - _S), (1, _SP - _S)))

    qn, kn, Q, K, V = pl.pallas_call(
        _stage_a,
        grid=grid,
        in_specs=[
            rowblk,
            full((_D, 5 * _D)), full((1, 5 * _D)),
            full((1, _D)), full((1, _D)),
        ],
        out_specs=[rowblk] * 5,
        out_shape=[bf3] * 5,
    )(x, Wall, ball, row2(ln1_g), row2(ln1_b))

    att = pl.pallas_call(
        _stage_bc,
        grid=grid,
        in_specs=[
            rowblk,
            seqblk,
            pl.BlockSpec((_QB, _SP), lambda b, i: (i, 0)),
            rowblk,
            seqblk,
            seqblk,
        ],
        out_specs=rowblk,
        out_shape=jax.ShapeDtypeStruct((_B, _SP, _D), _BF16),
    )(qn, kn, pos_pad, Q, K, V)

    out = pl.pallas_call(
        _stage_d,
        grid=grid,
        in_specs=[
            rowblk,
            rowblk,
            full((_D, _D)), full((1, _D)),
            full((1, _D)), full((1, _D)),
            full((_D, _MLP)), full((1, _MLP)),
            full((_MLP, _D)), full((1, _D)),
        ],
        out_specs=rowblk,
        out_shape=jax.ShapeDtypeStruct((_B, _S, _D), _F32),
    )(att, x, Wproj.astype(_BF16), row2(bproj), row2(ln2_g), row2(ln2_b),
      W1.astype(_BF16), row2(b1), W2.astype(_BF16), row2(b2))

    return out


# speedup vs baseline: 1.5582x; 1.0162x over previous
import jax
import jax.numpy as jnp
from jax.experimental import pallas as pl

_B = 4
_P = 576
_S = _P + 1
_D = 768
_H = 12
_HD = _D // _H
_K = 32
_TEMP = 0.1
_SCALE = _HD ** (-0.5)
_MLP = 4 * _D

_SP = 640

_F32 = jnp.float32
_BF16 = jnp.bfloat16


def _ln(x, g, b):
    mu = jnp.mean(x, axis=1, keepdims=True)
    var = jnp.mean((x - mu) ** 2, axis=1, keepdims=True)
    return (x - mu) / jnp.sqrt(var + 1e-5) * g + b


def _dot_t(a, b):
    return jax.lax.dot_general(a, b, (((1,), (1,)), ((), ())),
                               preferred_element_type=_F32)


def _body(x_ref, w_ref, bb_ref, g_ref, b_ref, pos_ref,
          wp_ref, bp_ref, g2_ref, b2_ref,
          w1_ref, b1_ref, w2_ref, bb2_ref, out_ref):
    rows1 = jax.lax.broadcasted_iota(jnp.int32, (_SP, 1), 0)
    xn = _ln(x_ref[...], g_ref[...], b_ref[...])
    xn = jnp.where(rows1 < _S, xn, 0.0)
    proj = jnp.dot(xn.astype(_BF16), w_ref[...],
                   preferred_element_type=_F32) + bb_ref[...]
    q = proj[:, :_D]
    qn = (q / jnp.maximum(
        jnp.sqrt(jnp.sum(q * q, axis=1, keepdims=True)), 1e-12)).astype(_BF16)
    k = proj[:, _D:2 * _D]
    kn = (k / jnp.maximum(
        jnp.sqrt(jnp.sum(k * k, axis=1, keepdims=True)), 1e-12)).astype(_BF16)
    Q = proj[:, 2 * _D:3 * _D].astype(_BF16)
    K = proj[:, 3 * _D:4 * _D].astype(_BF16)
    V = proj[:, 4 * _D:].astype(_BF16)

    rows = jax.lax.broadcasted_iota(jnp.int32, (_SP, _SP), 0)
    cols = jax.lax.broadcasted_iota(jnp.int32, (_SP, _SP), 1)
    rs = _dot_t(qn, kn) + pos_ref[...]
    valid = (cols >= 1) & (cols < _S) & (cols != rows)
    st = jnp.where(valid, rs * (1.0 / _TEMP), -1e30)
    m0 = jnp.max(st, axis=1, keepdims=True)
    lo = m0 - 23.0
    hi = m0
    for _ in range(16):
        t = 0.5 * (lo + hi)
        cnt = jnp.sum(jnp.where(st >= t, 1.0, 0.0), axis=1, keepdims=True)
        pred = cnt >= float(_K)
        lo = jnp.where(pred, t, lo)
        hi = jnp.where(pred, hi, t)
    sel = st >= lo
    e = jnp.where(sel, jnp.exp(st - m0), 0.0)
    z = jnp.sum(e, axis=1, keepdims=True)
    bias = jnp.where(sel, jnp.maximum(st - m0 - jnp.log(z), -10.0), -1e9)
    bias = jnp.where(rows == 0, jnp.where(cols < _S, 0.0, -1e9), bias)
    bias = jnp.where(rows >= _S, 0.0, bias)

    heads = []
    for h in range(_H):
        sl = slice(h * _HD, (h + 1) * _HD)
        s = _dot_t(Q[:, sl], K[:, sl]) * _SCALE + bias
        m = jnp.max(s, axis=1, keepdims=True)
        p = jnp.exp(s - m)
        zz = jnp.sum(p, axis=1, keepdims=True)
        heads.append((jnp.dot(p.astype(_BF16), V[:, sl],
                              preferred_element_type=_F32) / zz
                      ).astype(_BF16))
    ao = jnp.concatenate(heads, axis=1)

    hh = jnp.dot(ao, wp_ref[...],
                 preferred_element_type=_F32) + bp_ref[...] + x_ref[...]
    hn = _ln(hh, g2_ref[...], b2_ref[...])
    u = jnp.dot(hn.astype(_BF16), w1_ref[...],
                preferred_element_type=_F32) + b1_ref[...]
    gelu = 0.5 * u * (1.0 + jax.lax.erf(u * (2.0 ** -0.5)))
    out_ref[...] = hh + jnp.dot(gelu.astype(_BF16), w2_ref[...],
                                preferred_element_type=_F32) + bb2_ref[...]


def kernel(x, Wq, bq, Wk, bk, pos_bias, Wqkv, bqkv, Wproj, bproj,
           ln1_g, ln1_b, ln2_g, ln2_b, W1, b1, W2, b2):
    row2 = lambda a: a.reshape(1, -1)
    full = lambda shape: pl.BlockSpec(shape, lambda *_: (0,) * len(shape))
    rowblk = pl.BlockSpec((None, _SP, _D), lambda b: (b, 0, 0))

    Wall = jnp.concatenate([Wq, Wk, Wqkv], axis=1).astype(_BF16)
    ball = jnp.concatenate([bq, bk, bqkv]).reshape(1, -1)
    pos_pad = jnp.pad(pos_bias, ((1, _SP - _S), (1, _SP - _S)))

    out = pl.pallas_call(
        _body,
        grid=(_B,),
        in_specs=[
            rowblk,
            full((_D, 5 * _D)), full((1, 5 * _D)),
            full((1, _D)), full((1, _D)),
            full((_SP, _SP)),
            full((_D, _D)), full((1, _D)),
            full((1, _D)), full((1, _D)),
            full((_D, _MLP)), full((1, _MLP)),
            full((_MLP, _D)), full((1, _D)),
        ],
        out_specs=rowblk,
        out_shape=jax.ShapeDtypeStruct((_B, _S, _D), _F32),
    )(x, Wall, ball, row2(ln1_g), row2(ln1_b), pos_pad,
      Wproj.astype(_BF16), row2(bproj), row2(ln2_g), row2(ln2_b),
      W1.astype(_BF16), row2(b1), W2.astype(_BF16), row2(b2))

    return out
